# baseline (speedup 1.0000x reference)
import functools

import jax
import jax.numpy as jnp
from jax import lax
from jax.experimental import pallas as pl
from jax.experimental.pallas import tpu as pltpu
from jax.experimental.pallas import tpu_sc as plsc

N_NODES = 10000
N_EDGES = 320000
IN_DIM = 128
HID_DIM = 128
OUT_DIM = 16

NC, NS = 2, 16
NW = NC * NS
CHUNK = 64
NP = 10240
ROWS_PER_TILE = NP // NS
ZR = 16

E_PER_TILE_CHUNKS = 160
E_PER_TILE = E_PER_TILE_CHUNKS * CHUNK
E_PAD = E_PER_TILE * NW
PAD_DST = 10100

_MESH = plsc.VectorSubcoreMesh(core_axis_name="c", subcore_axis_name="s")


def _zero_fill(zbuf, rows, d):
    for r in range(rows):
        for j in range(d // 16):
            zbuf[r, pl.ds(j * 16, 16)] = jnp.zeros((16,), jnp.float32)


NCH = E_PER_TILE_CHUNKS
IDXB = 16
NBUF = 4

NCH0 = 320
NCH1 = 0
assert NCH0 + NCH1 == 2 * NCH
assert NCH0 % IDXB == 0 and NCH1 % IDXB == 0 and IDXB % NBUF == 0


def _make_edge_scatter(d):
    @functools.partial(
        pl.kernel,
        out_type=jax.ShapeDtypeStruct((NC, NP, d), jnp.float32),
        mesh=_MESH,
        scratch_types=[
            pltpu.VMEM_SHARED((NP, d), jnp.float32),
            pltpu.VMEM((IDXB, CHUNK), jnp.int32),
            pltpu.VMEM((IDXB, CHUNK), jnp.int32),
            [pltpu.VMEM((CHUNK, d), jnp.float32) for _ in range(NBUF)],
            [pltpu.SemaphoreType.DMA for _ in range(NBUF)],
        ],
    )
    def k(g_hbm, src_hbm, dst_hbm, out_hbm, acc, idx_s, idx_d, bufs, sems):
        c = lax.axis_index("c")
        s = lax.axis_index("s")
        crow = jnp.where(c == 0, s * NCH0, NS * NCH0 + s * NCH1)
        nblocks = jnp.where(c == 0, NCH0 // IDXB, NCH1 // IDXB)
        _zero_fill(bufs[0], CHUNK, d)
        row0 = s * ROWS_PER_TILE

        def zloop(i, carry):
            pltpu.sync_copy(bufs[0], acc.at[pl.ds(row0 + i * CHUNK, CHUNK)])
            return carry

        lax.fori_loop(0, ROWS_PER_TILE // CHUNK, zloop, 0)
        plsc.subcore_barrier()

        def bloop(bi, carry):
            pltpu.sync_copy(src_hbm.at[pl.ds(crow + bi * IDXB, IDXB)], idx_s)
            pltpu.sync_copy(dst_hbm.at[pl.ds(crow + bi * IDXB, IDXB)], idx_d)
            for p in range(NBUF - 1):
                pltpu.async_copy(g_hbm.at[idx_s.at[p]], bufs[p], sems[p])

            def eloop(cio, carry2):
                for b in range(NBUF):
                    ci = cio * NBUF + b
                    nx = ci + NBUF - 1
                    nb = (b + NBUF - 1) % NBUF

                    @pl.when(nx < IDXB)
                    def _():
                        pltpu.async_copy(
                            g_hbm.at[idx_s.at[nx]], bufs[nb], sems[nb])

                    pltpu.make_async_copy(
                        g_hbm.at[idx_s.at[ci]], bufs[b], sems[b]).wait()
                    pltpu.sync_copy(bufs[b], acc.at[idx_d.at[ci]], add=True)
                return carry2

            lax.fori_loop(0, IDXB // NBUF, eloop, 0)
            return carry

        lax.fori_loop(0, nblocks, bloop, 0)
        plsc.subcore_barrier()
        pltpu.sync_copy(
            acc.at[pl.ds(row0, ROWS_PER_TILE)],
            out_hbm.at[c].at[pl.ds(row0, ROWS_PER_TILE)],
        )

    return k


_edge_scatter_hid = _make_edge_scatter(HID_DIM)

DEG_W = 128


@functools.partial(
    pl.kernel,
    out_type=jax.ShapeDtypeStruct((NC, NP, DEG_W), jnp.float32),
    mesh=_MESH,
    scratch_types=[
        pltpu.VMEM_SHARED((NP, DEG_W), jnp.float32),
        pltpu.VMEM((NCH, CHUNK), jnp.int32),
        pltpu.VMEM((CHUNK, DEG_W), jnp.float32),
        pltpu.VMEM((ZR, DEG_W), jnp.float32),
    ],
)
def _degree_kernel(dst_hbm, out_hbm, acc, idx_d, ones, zbuf):
    c = lax.axis_index("c")
    s = lax.axis_index("s")
    wid = s * NC + c
    pltpu.sync_copy(dst_hbm.at[pl.ds(wid * NCH, NCH)], idx_d)
    _zero_fill(zbuf, ZR, DEG_W)

    def ones_fill(r, carry):
        for j in range(DEG_W // 16):
            ones[r, pl.ds(j * 16, 16)] = jnp.ones((16,), jnp.float32)
        return carry

    lax.fori_loop(0, CHUNK, ones_fill, 0)
    row0 = s * ROWS_PER_TILE

    def zloop(i, carry):
        pltpu.sync_copy(zbuf, acc.at[pl.ds(row0 + i * ZR, ZR)])
        return carry

    lax.fori_loop(0, ROWS_PER_TILE // ZR, zloop, 0)
    plsc.subcore_barrier()

    def eloop(ci, carry):
        pltpu.sync_copy(ones, acc.at[idx_d.at[ci]], add=True)
        return carry

    lax.fori_loop(0, NCH, eloop, 0)
    plsc.subcore_barrier()
    pltpu.sync_copy(
        acc.at[pl.ds(row0, ROWS_PER_TILE)],
        out_hbm.at[c].at[pl.ds(row0, ROWS_PER_TILE)],
    )


TC_BLK = 1280


def _tc_a_body(x_ref, w_ref, da_ref, db_ref, o_ref):
    deg = da_ref[...] + db_ref[...] + 1.0
    dinv = lax.rsqrt(deg)
    h = jnp.dot(x_ref[...], w_ref[...], preferred_element_type=jnp.float32)
    o_ref[...] = h * dinv


def _tc_b_body(sa_ref, sb_ref, g_ref, da_ref, db_ref, b_ref, o_ref):
    deg = da_ref[...] + db_ref[...] + 1.0
    dinv = lax.rsqrt(deg)
    z = dinv * (sa_ref[...] + sb_ref[...] + g_ref[...]) + b_ref[...]
    o_ref[...] = dinv * jnp.maximum(z, 0.0)


def _tc_c_body(ta_ref, tb_ref, u_ref, da_ref, db_ref, b_ref, w_ref, o_ref):
    deg = da_ref[...] + db_ref[...] + 1.0
    dinv = lax.rsqrt(deg)
    t = ta_ref[...] + tb_ref[...] + u_ref[...]
    h = jnp.dot(t, w_ref[...], preferred_element_type=jnp.float32)
    o_ref[...] = dinv * h + b_ref[...]


def _row_spec(d):
    return pl.BlockSpec((TC_BLK, d), lambda i: (i, 0))


def _full_spec(a, b):
    return pl.BlockSpec((a, b), lambda i: (0, 0))


_GRID = NP // TC_BLK

_tc_a = pl.pallas_call(
    _tc_a_body,
    grid=(_GRID,),
    in_specs=[
        _row_spec(IN_DIM),
        _full_spec(IN_DIM, HID_DIM),
        _row_spec(1),
        _row_spec(1),
    ],
    out_specs=_row_spec(HID_DIM),
    out_shape=jax.ShapeDtypeStruct((NP, HID_DIM), jnp.float32),
)

_tc_b = pl.pallas_call(
    _tc_b_body,
    grid=(_GRID,),
    in_specs=[
        _row_spec(HID_DIM),
        _row_spec(HID_DIM),
        _row_spec(HID_DIM),
        _row_spec(1),
        _row_spec(1),
        _full_spec(1, HID_DIM),
    ],
    out_specs=_row_spec(HID_DIM),
    out_shape=jax.ShapeDtypeStruct((NP, HID_DIM), jnp.float32),
)

_tc_c = pl.pallas_call(
    _tc_c_body,
    grid=(_GRID,),
    in_specs=[
        _row_spec(HID_DIM),
        _row_spec(HID_DIM),
        _row_spec(HID_DIM),
        _row_spec(1),
        _row_spec(1),
        _full_spec(1, OUT_DIM),
        _full_spec(HID_DIM, OUT_DIM),
    ],
    out_specs=_row_spec(OUT_DIM),
    out_shape=jax.ShapeDtypeStruct((NP, OUT_DIM), jnp.float32),
)


@jax.jit
def kernel(x, train_pos_edge_index, W1, b1, W2, b2):
    ei = train_pos_edge_index.astype(jnp.int32)
    pad = E_PAD - N_EDGES
    src = jnp.concatenate(
        [ei[0], jnp.zeros((pad,), jnp.int32)]).reshape(NW * NCH, CHUNK)
    dst = jnp.concatenate(
        [ei[1], jnp.full((pad,), PAD_DST, jnp.int32)]).reshape(NW * NCH, CHUNK)

    x_p = jnp.zeros((NP, IN_DIM), jnp.float32).at[:N_NODES].set(x)

    deg = _degree_kernel(dst)
    da = deg[0, :, 0:1]
    db = deg[1, :, 0:1]

    g1 = _tc_a(x_p, W1, da, db)
    s1 = _edge_scatter_hid(g1, src, dst)
    u = _tc_b(s1[0], s1[1], g1, da, db, b1.reshape(1, HID_DIM))
    t = _edge_scatter_hid(u, src, dst)
    out = _tc_c(t[0], t[1], u, da, db, b2.reshape(1, OUT_DIM), W2)
    return out[:N_NODES]


# speedup vs baseline: 1.2080x; 1.2080x over previous
import functools

import jax
import jax.numpy as jnp
from jax import lax
from jax.experimental import pallas as pl
from jax.experimental.pallas import tpu as pltpu
from jax.experimental.pallas import tpu_sc as plsc

N_NODES = 10000
N_EDGES = 320000
IN_DIM = 128
HID_DIM = 128
OUT_DIM = 16

NC, NS = 2, 16
NW = NC * NS
CHUNK = 128
NP = 10240
ROWS_PER_TILE = NP // NS
ZR = 16

E_PER_TILE_CHUNKS = 80
E_PER_TILE = E_PER_TILE_CHUNKS * CHUNK
E_PAD = E_PER_TILE * NW
PAD_DST = 10100

NCH = E_PER_TILE_CHUNKS
IDXB = 16

NCH0 = 144
NCH1 = 16
assert NCH0 + NCH1 == 2 * NCH and NCH0 % IDXB == 0 and NCH1 % IDXB == 0

_MESH = plsc.VectorSubcoreMesh(core_axis_name="c", subcore_axis_name="s")


def _zero_fill(zbuf, rows, d):
    for r in range(rows):
        for j in range(d // 16):
            zbuf[r, pl.ds(j * 16, 16)] = jnp.zeros((16,), jnp.float32)


def _make_edge_scatter(d):
    @functools.partial(
        pl.kernel,
        out_type=jax.ShapeDtypeStruct((NC, NP, d), jnp.float32),
        mesh=_MESH,
        scratch_types=[
            pltpu.VMEM_SHARED((NP, d), jnp.float32),
            pltpu.VMEM((IDXB, CHUNK), jnp.int32),
            pltpu.VMEM((IDXB, CHUNK), jnp.int32),
            pltpu.VMEM((CHUNK, d), jnp.float32),
            pltpu.VMEM((CHUNK, d), jnp.float32),
            pltpu.VMEM((ZR, d), jnp.float32),
            pltpu.SemaphoreType.DMA,
            pltpu.SemaphoreType.DMA,
        ],
    )
    def k(g_hbm, src_hbm, dst_hbm, out_hbm, acc, idx_s, idx_d,
          rows0, rows1, zbuf, sem0, sem1):
        c = lax.axis_index("c")
        s = lax.axis_index("s")
        crow = jnp.where(c == 0, s * NCH0, NS * NCH0 + s * NCH1)
        nblocks = jnp.where(c == 0, NCH0 // IDXB, NCH1 // IDXB)
        _zero_fill(zbuf, ZR, d)
        row0 = s * ROWS_PER_TILE

        def zloop(i, carry):
            pltpu.sync_copy(zbuf, acc.at[pl.ds(row0 + i * ZR, ZR)])
            return carry

        lax.fori_loop(0, ROWS_PER_TILE // ZR, zloop, 0)
        plsc.subcore_barrier()

        bufs = (rows0, rows1)
        sems = (sem0, sem1)

        def bloop(bi, carry):
            pltpu.sync_copy(src_hbm.at[pl.ds(crow + bi * IDXB, IDXB)], idx_s)
            pltpu.sync_copy(dst_hbm.at[pl.ds(crow + bi * IDXB, IDXB)], idx_d)
            pltpu.async_copy(g_hbm.at[idx_s.at[0]], rows0, sem0)

            def eloop(cio, carry2):
                for b in range(2):
                    ci = cio * 2 + b

                    @pl.when(ci + 1 < IDXB)
                    def _():
                        pltpu.async_copy(
                            g_hbm.at[idx_s.at[ci + 1]], bufs[1 - b],
                            sems[1 - b])

                    pltpu.make_async_copy(
                        g_hbm.at[idx_s.at[ci]], bufs[b], sems[b]).wait()
                    pltpu.sync_copy(bufs[b], acc.at[idx_d.at[ci]], add=True)
                return carry2

            lax.fori_loop(0, IDXB // 2, eloop, 0)
            return carry

        lax.fori_loop(0, nblocks, bloop, 0)
        plsc.subcore_barrier()
        pltpu.sync_copy(
            acc.at[pl.ds(row0, ROWS_PER_TILE)],
            out_hbm.at[c].at[pl.ds(row0, ROWS_PER_TILE)],
        )

    return k


_edge_scatter_hid = _make_edge_scatter(HID_DIM)

DEG_W = 128


@functools.partial(
    pl.kernel,
    out_type=jax.ShapeDtypeStruct((NC, NP, DEG_W), jnp.float32),
    mesh=_MESH,
    scratch_types=[
        pltpu.VMEM_SHARED((NP, DEG_W), jnp.float32),
        pltpu.VMEM((NCH, CHUNK), jnp.int32),
        pltpu.VMEM((CHUNK, DEG_W), jnp.float32),
        pltpu.VMEM((ZR, DEG_W), jnp.float32),
    ],
)
def _degree_kernel(dst_hbm, out_hbm, acc, idx_d, ones, zbuf):
    c = lax.axis_index("c")
    s = lax.axis_index("s")
    wid = s * NC + c
    pltpu.sync_copy(dst_hbm.at[pl.ds(wid * NCH, NCH)], idx_d)
    _zero_fill(zbuf, ZR, DEG_W)

    def ones_fill(r, carry):
        for j in range(DEG_W // 16):
            ones[r, pl.ds(j * 16, 16)] = jnp.ones((16,), jnp.float32)
        return carry

    lax.fori_loop(0, CHUNK, ones_fill, 0)
    row0 = s * ROWS_PER_TILE

    def zloop(i, carry):
        pltpu.sync_copy(zbuf, acc.at[pl.ds(row0 + i * ZR, ZR)])
        return carry

    lax.fori_loop(0, ROWS_PER_TILE // ZR, zloop, 0)
    plsc.subcore_barrier()

    def eloop(ci, carry):
        pltpu.sync_copy(ones, acc.at[idx_d.at[ci]], add=True)
        return carry

    lax.fori_loop(0, NCH, eloop, 0)
    plsc.subcore_barrier()
    pltpu.sync_copy(
        acc.at[pl.ds(row0, ROWS_PER_TILE)],
        out_hbm.at[c].at[pl.ds(row0, ROWS_PER_TILE)],
    )


TC_BLK = 1280


def _tc_a_body(x_ref, w_ref, da_ref, db_ref, o_ref):
    deg = da_ref[...] + db_ref[...] + 1.0
    dinv = lax.rsqrt(deg)
    h = jnp.dot(x_ref[...], w_ref[...], preferred_element_type=jnp.float32)
    o_ref[...] = h * dinv


def _tc_b_body(sa_ref, sb_ref, g_ref, da_ref, db_ref, b_ref, o_ref):
    deg = da_ref[...] + db_ref[...] + 1.0
    dinv = lax.rsqrt(deg)
    z = dinv * (sa_ref[...] + sb_ref[...] + g_ref[...]) + b_ref[...]
    o_ref[...] = dinv * jnp.maximum(z, 0.0)


def _tc_c_body(ta_ref, tb_ref, u_ref, da_ref, db_ref, b_ref, w_ref, o_ref):
    deg = da_ref[...] + db_ref[...] + 1.0
    dinv = lax.rsqrt(deg)
    t = ta_ref[...] + tb_ref[...] + u_ref[...]
    h = jnp.dot(t, w_ref[...], preferred_element_type=jnp.float32)
    o_ref[...] = dinv * h + b_ref[...]


def _row_spec(d):
    return pl.BlockSpec((TC_BLK, d), lambda i: (i, 0))


def _full_spec(a, b):
    return pl.BlockSpec((a, b), lambda i: (0, 0))


_GRID = NP // TC_BLK

_tc_a = pl.pallas_call(
    _tc_a_body,
    grid=(_GRID,),
    in_specs=[
        _row_spec(IN_DIM),
        _full_spec(IN_DIM, HID_DIM),
        _row_spec(1),
        _row_spec(1),
    ],
    out_specs=_row_spec(HID_DIM),
    out_shape=jax.ShapeDtypeStruct((NP, HID_DIM), jnp.float32),
)

_tc_b = pl.pallas_call(
    _tc_b_body,
    grid=(_GRID,),
    in_specs=[
        _row_spec(HID_DIM),
        _row_spec(HID_DIM),
        _row_spec(HID_DIM),
        _row_spec(1),
        _row_spec(1),
        _full_spec(1, HID_DIM),
    ],
    out_specs=_row_spec(HID_DIM),
    out_shape=jax.ShapeDtypeStruct((NP, HID_DIM), jnp.float32),
)

_tc_c = pl.pallas_call(
    _tc_c_body,
    grid=(_GRID,),
    in_specs=[
        _row_spec(HID_DIM),
        _row_spec(HID_DIM),
        _row_spec(HID_DIM),
        _row_spec(1),
        _row_spec(1),
        _full_spec(1, OUT_DIM),
        _full_spec(HID_DIM, OUT_DIM),
    ],
    out_specs=_row_spec(OUT_DIM),
    out_shape=jax.ShapeDtypeStruct((NP, OUT_DIM), jnp.float32),
)


@jax.jit
def kernel(x, train_pos_edge_index, W1, b1, W2, b2):
    ei = train_pos_edge_index.astype(jnp.int32)
    pad = E_PAD - N_EDGES
    src = jnp.concatenate(
        [ei[0], jnp.zeros((pad,), jnp.int32)]).reshape(NW * NCH, CHUNK)
    dst = jnp.concatenate(
        [ei[1], jnp.full((pad,), PAD_DST, jnp.int32)]).reshape(NW * NCH, CHUNK)

    x_p = jnp.zeros((NP, IN_DIM), jnp.float32).at[:N_NODES].set(x)

    deg = _degree_kernel(dst)
    da = deg[0, :, 0:1]
    db = deg[1, :, 0:1]

    g1 = _tc_a(x_p, W1, da, db)
    s1 = _edge_scatter_hid(g1, src, dst)
    u = _tc_b(s1[0], s1[1], g1, da, db, b1.reshape(1, HID_DIM))
    t = _edge_scatter_hid(u, src, dst)
    out = _tc_c(t[0], t[1], u, da, db, b2.reshape(1, OUT_DIM), W2)
    return out[:N_NODES]
